# Initial kernel scaffold; baseline (speedup 1.0000x reference)
import jax
import jax.numpy as jnp
from jax.experimental import pallas as pl


def kernel(x, edge_index, batch, W_gnn, b_gnn, ln_scale, ln_bias, W_pol, b_pol, W_val, b_val):
    raise NotImplementedError("write your pallas kernel here")


# speedup vs baseline: 4.0463x; 4.0463x over previous
import jax
import jax.numpy as jnp
from jax import lax
from jax.experimental import pallas as pl
from jax.experimental.pallas import tpu as pltpu
from jax.experimental.pallas import tpu_sc as plsc

N = 10000
D = 128
H = 128
S = 64
BH = 32

NC = 2
NS = 16
LANES = 128
BLK_PER_TILE = 80
EDGES_PER_TILE = BLK_PER_TILE * LANES
E_PAD = NC * NS * EDGES_PER_TILE
NP = 10240
ROWS_PER_TILE = NP // NS

TC_BLK = 1024
TC_STEPS = NP // TC_BLK


def _sc_edge_body(x_hbm, src_hbm, dst_hbm, zrows_hbm, zdeg_hbm,
                  out_hbm, outd_hbm,
                  src_idx, dst_idx, rows, deg, acc, sem):
    c = lax.axis_index("c")
    s = lax.axis_index("s")
    pltpu.sync_copy(zrows_hbm, acc.at[pl.ds(s * ROWS_PER_TILE, ROWS_PER_TILE)])
    pltpu.sync_copy(zdeg_hbm, deg)
    blk_base = (c * NS + s) * BLK_PER_TILE
    pltpu.sync_copy(src_hbm.at[pl.ds(blk_base, BLK_PER_TILE)], src_idx)
    pltpu.sync_copy(dst_hbm.at[pl.ds(blk_base, BLK_PER_TILE)], dst_idx)
    plsc.subcore_barrier()

    ones16 = jnp.full((16,), 1.0, jnp.float32)

    @pl.loop(0, BLK_PER_TILE)
    def _edge_block(j):
        cp = pltpu.async_copy(x_hbm.at[src_idx.at[j]], rows, sem)
        for k in range(LANES // 16):
            vals = dst_idx[j, pl.ds(k * 16, 16)]
            plsc.addupdate_scatter(deg, [vals], ones16)
        cp.wait()
        pltpu.sync_copy(rows, acc.at[dst_idx.at[j]], add=True)

    plsc.subcore_barrier()
    pltpu.sync_copy(acc.at[pl.ds(s * ROWS_PER_TILE, ROWS_PER_TILE)],
                    out_hbm.at[c].at[pl.ds(s * ROWS_PER_TILE, ROWS_PER_TILE)])
    pltpu.sync_copy(deg, outd_hbm.at[c].at[s])


def _tc_tail_body(acc_ref, deg_ref, wg_ref, bg_ref, lns_ref, lnb_ref,
                  wp_ref, bp_ref, wv_ref, bv_ref,
                  logits_ref, value_ref, psum):
    i = pl.program_id(0)
    st = acc_ref[0] + acc_ref[1]
    deg = jnp.sum(deg_ref[...], axis=(0, 1)).reshape(TC_BLK, 1)
    aw = st / jnp.maximum(deg, 1.0)
    h = jnp.dot(aw, wg_ref[...], preferred_element_type=jnp.float32) + bg_ref[...]
    h = jnp.maximum(h, 0.0)
    row_id = i * TC_BLK + lax.broadcasted_iota(jnp.int32, (TC_BLK, 1), 0)
    h = jnp.where(row_id < N, h, 0.0)
    part = jnp.sum(h, axis=0, keepdims=True)

    @pl.when(i == 0)
    def _init():
        psum[...] = part

    @pl.when(i > 0)
    def _accum():
        psum[...] = psum[...] + part

    @pl.when(i == TC_STEPS - 1)
    def _finish():
        pooled = psum[...] * (1.0 / N)
        mu = jnp.mean(pooled, axis=1, keepdims=True)
        var = jnp.mean((pooled - mu) ** 2, axis=1, keepdims=True)
        nrm = (pooled - mu) / jnp.sqrt(var + 1e-5) * lns_ref[...] + lnb_ref[...]
        logits_ref[...] = (
            jnp.dot(nrm, wp_ref[...], preferred_element_type=jnp.float32)
            + bp_ref[...])
        value_ref[...] = (
            jnp.dot(nrm, wv_ref[...], preferred_element_type=jnp.float32)
            + bv_ref[...])


def kernel(x, edge_index, batch, W_gnn, b_gnn, ln_scale, ln_bias,
           W_pol, b_pol, W_val, b_val):
    f32 = jnp.float32
    n_pad = E_PAD - edge_index.shape[1]
    src = jnp.concatenate([edge_index[0], jnp.zeros((n_pad,), jnp.int32)])
    dst = jnp.concatenate([edge_index[1], jnp.full((n_pad,), N, jnp.int32)])
    src2d = src.reshape(NC * NS * BLK_PER_TILE, LANES)
    dst2d = dst.reshape(NC * NS * BLK_PER_TILE, LANES)
    zrows = jnp.zeros((ROWS_PER_TILE, H), f32)
    zdeg = jnp.zeros((NP,), f32)

    sc_edge = pl.kernel(
        _sc_edge_body,
        out_type=[
            jax.ShapeDtypeStruct((NC, NP, H), f32),
            jax.ShapeDtypeStruct((NC, NS, NP), f32),
        ],
        mesh=plsc.VectorSubcoreMesh(
            core_axis_name="c", subcore_axis_name="s",
            num_cores=NC, num_subcores=NS),
        compiler_params=pltpu.CompilerParams(needs_layout_passes=False),
        scratch_types=[
            pltpu.VMEM((BLK_PER_TILE, LANES), jnp.int32),
            pltpu.VMEM((BLK_PER_TILE, LANES), jnp.int32),
            pltpu.VMEM((LANES, H), f32),
            pltpu.VMEM((NP,), f32),
            pltpu.VMEM_SHARED((NP, H), f32),
            pltpu.SemaphoreType.DMA,
        ],
    )
    acc2, deg2 = sc_edge(x, src2d, dst2d, zrows, zdeg)

    grid = (TC_STEPS,)
    logits2d, value2d = pl.pallas_call(
        _tc_tail_body,
        grid=grid,
        in_specs=[
            pl.BlockSpec((NC, TC_BLK, H), lambda i: (0, i, 0)),
            pl.BlockSpec((NC, NS, TC_BLK), lambda i: (0, 0, i)),
            pl.BlockSpec((H, H), lambda i: (0, 0)),
            pl.BlockSpec((1, H), lambda i: (0, 0)),
            pl.BlockSpec((1, H), lambda i: (0, 0)),
            pl.BlockSpec((1, H), lambda i: (0, 0)),
            pl.BlockSpec((H, S * BH), lambda i: (0, 0)),
            pl.BlockSpec((1, S * BH), lambda i: (0, 0)),
            pl.BlockSpec((H, 1), lambda i: (0, 0)),
            pl.BlockSpec((1, 1), lambda i: (0, 0)),
        ],
        out_specs=[
            pl.BlockSpec((1, S * BH), lambda i: (0, 0)),
            pl.BlockSpec((1, 1), lambda i: (0, 0)),
        ],
        out_shape=[
            jax.ShapeDtypeStruct((1, S * BH), f32),
            jax.ShapeDtypeStruct((1, 1), f32),
        ],
        scratch_shapes=[pltpu.VMEM((1, H), f32)],
    )(acc2, deg2, W_gnn, b_gnn.reshape(1, H), ln_scale.reshape(1, H),
      ln_bias.reshape(1, H), W_pol, b_pol.reshape(1, S * BH),
      W_val, b_val.reshape(1, 1))

    logits = logits2d.reshape(1, S, BH)
    value = value2d.reshape(1)
    return (logits, value)


# speedup vs baseline: 4.4638x; 1.1032x over previous
import jax
import jax.numpy as jnp
from jax import lax
from jax.experimental import pallas as pl
from jax.experimental.pallas import tpu as pltpu
from jax.experimental.pallas import tpu_sc as plsc

N = 10000
D = 128
H = 128
S = 64
BH = 32

NC = 2
NS = 16
GB = 128
CH = 4
BLK_PER_TILE = 80
NCH = BLK_PER_TILE // CH
EDGES_PER_TILE = BLK_PER_TILE * GB
E_PAD = NC * NS * EDGES_PER_TILE
NP = 10240
ROWS_PER_TILE = NP // NS

TC_BLK = 1024
TC_STEPS = NP // TC_BLK


def _sc_edge_body(x_hbm, src_hbm, dst_hbm, zrows_hbm, zdeg_hbm,
                  out_hbm, outd_hbm,
                  src_idx, dst_idx, rows0, rows1, deg, acc, sg0, sg1, si):
    c = lax.axis_index("c")
    s = lax.axis_index("s")
    rows = (rows0, rows1)
    sg = (sg0, sg1)
    pltpu.sync_copy(zrows_hbm, acc.at[pl.ds(s * ROWS_PER_TILE, ROWS_PER_TILE)])
    pltpu.sync_copy(zdeg_hbm, deg)

    blk_base = (c * NS + s) * BLK_PER_TILE

    def _refill(buf, ch, issue):
        for hbm_ref, vref in ((src_hbm, src_idx), (dst_hbm, dst_idx)):
            cp = pltpu.make_async_copy(
                hbm_ref.at[pl.ds(blk_base + ch * CH, CH)], vref.at[buf], si)
            cp.start() if issue else cp.wait()

    ones16 = jnp.full((16,), 1.0, jnp.float32)

    pltpu.sync_copy(src_hbm.at[pl.ds(blk_base, CH)], src_idx.at[0])
    pltpu.sync_copy(dst_hbm.at[pl.ds(blk_base, CH)], dst_idx.at[0])
    _refill(1, 1, True)
    plsc.subcore_barrier()
    pltpu.async_copy(x_hbm.at[src_idx.at[0].at[0]], rows0, sg0)

    def _chunk(b, ch):
        for jj in range(CH):
            cur = jj % 2
            nxt = 1 - cur
            if jj < CH - 1:
                pltpu.async_copy(x_hbm.at[src_idx.at[b].at[jj + 1]],
                                 rows[nxt], sg[nxt])
            else:
                @pl.when(ch + 1 < NCH)
                def _pref():
                    _refill(1 - b, ch + 1, False)
                    pltpu.async_copy(x_hbm.at[src_idx.at[1 - b].at[0]],
                                     rows[nxt], sg[nxt])
            for k in range(GB // 16):
                vals = dst_idx[b, jj, pl.ds(k * 16, 16)]
                plsc.addupdate_scatter(deg, [vals], ones16)
            pltpu.make_async_copy(x_hbm.at[src_idx.at[b].at[jj]],
                                  rows[cur], sg[cur]).wait()
            pltpu.sync_copy(rows[cur], acc.at[dst_idx.at[b].at[jj]], add=True)
        @pl.when(ch + 2 < NCH)
        def _next_refill():
            _refill(b, ch + 2, True)

    @pl.loop(0, NCH, step=2)
    def _chunk_pair(ch):
        _chunk(0, ch)
        _chunk(1, ch + 1)

    plsc.subcore_barrier()
    pltpu.sync_copy(acc.at[pl.ds(s * ROWS_PER_TILE, ROWS_PER_TILE)],
                    out_hbm.at[c].at[pl.ds(s * ROWS_PER_TILE, ROWS_PER_TILE)])
    pltpu.sync_copy(deg, outd_hbm.at[c].at[s])


def _tc_tail_body(acc_ref, deg_ref, wg_ref, bg_ref, lns_ref, lnb_ref,
                  wp_ref, bp_ref, wv_ref, bv_ref,
                  logits_ref, value_ref, psum):
    i = pl.program_id(0)
    st = acc_ref[0] + acc_ref[1]
    deg = jnp.sum(deg_ref[...], axis=(0, 1)).reshape(TC_BLK, 1)
    aw = st / jnp.maximum(deg, 1.0)
    h = jnp.dot(aw, wg_ref[...], preferred_element_type=jnp.float32) + bg_ref[...]
    h = jnp.maximum(h, 0.0)
    row_id = i * TC_BLK + lax.broadcasted_iota(jnp.int32, (TC_BLK, 1), 0)
    h = jnp.where(row_id < N, h, 0.0)
    part = jnp.sum(h, axis=0, keepdims=True)

    @pl.when(i == 0)
    def _init():
        psum[...] = part

    @pl.when(i > 0)
    def _accum():
        psum[...] = psum[...] + part

    @pl.when(i == TC_STEPS - 1)
    def _finish():
        pooled = psum[...] * (1.0 / N)
        mu = jnp.mean(pooled, axis=1, keepdims=True)
        var = jnp.mean((pooled - mu) ** 2, axis=1, keepdims=True)
        nrm = (pooled - mu) / jnp.sqrt(var + 1e-5) * lns_ref[...] + lnb_ref[...]
        logits_ref[...] = (
            jnp.dot(nrm, wp_ref[...], preferred_element_type=jnp.float32)
            + bp_ref[...])
        value_ref[...] = (
            jnp.dot(nrm, wv_ref[...], preferred_element_type=jnp.float32)
            + bv_ref[...])


def kernel(x, edge_index, batch, W_gnn, b_gnn, ln_scale, ln_bias,
           W_pol, b_pol, W_val, b_val):
    f32 = jnp.float32
    n_pad = E_PAD - edge_index.shape[1]
    src = jnp.concatenate([edge_index[0], jnp.zeros((n_pad,), jnp.int32)])
    dst = jnp.concatenate([edge_index[1], jnp.full((n_pad,), N, jnp.int32)])
    src2d = src.reshape(NC * NS * BLK_PER_TILE, GB)
    dst2d = dst.reshape(NC * NS * BLK_PER_TILE, GB)
    zrows = jnp.zeros((ROWS_PER_TILE, H), f32)
    zdeg = jnp.zeros((NP,), f32)

    sc_edge = pl.kernel(
        _sc_edge_body,
        out_type=[
            jax.ShapeDtypeStruct((NC, NP, H), f32),
            jax.ShapeDtypeStruct((NC, NS, NP), f32),
        ],
        mesh=plsc.VectorSubcoreMesh(
            core_axis_name="c", subcore_axis_name="s",
            num_cores=NC, num_subcores=NS),
        compiler_params=pltpu.CompilerParams(needs_layout_passes=False),
        scratch_types=[
            pltpu.VMEM((2, CH, GB), jnp.int32),
            pltpu.VMEM((2, CH, GB), jnp.int32),
            pltpu.VMEM((GB, H), f32),
            pltpu.VMEM((GB, H), f32),
            pltpu.VMEM((NP,), f32),
            pltpu.VMEM_SHARED((NP, H), f32),
            pltpu.SemaphoreType.DMA,
            pltpu.SemaphoreType.DMA,
            pltpu.SemaphoreType.DMA,
        ],
    )
    acc2, deg2 = sc_edge(x, src2d, dst2d, zrows, zdeg)

    grid = (TC_STEPS,)
    logits2d, value2d = pl.pallas_call(
        _tc_tail_body,
        grid=grid,
        in_specs=[
            pl.BlockSpec((NC, TC_BLK, H), lambda i: (0, i, 0)),
            pl.BlockSpec((NC, NS, TC_BLK), lambda i: (0, 0, i)),
            pl.BlockSpec((H, H), lambda i: (0, 0)),
            pl.BlockSpec((1, H), lambda i: (0, 0)),
            pl.BlockSpec((1, H), lambda i: (0, 0)),
            pl.BlockSpec((1, H), lambda i: (0, 0)),
            pl.BlockSpec((H, S * BH), lambda i: (0, 0)),
            pl.BlockSpec((1, S * BH), lambda i: (0, 0)),
            pl.BlockSpec((H, 1), lambda i: (0, 0)),
            pl.BlockSpec((1, 1), lambda i: (0, 0)),
        ],
        out_specs=[
            pl.BlockSpec((1, S * BH), lambda i: (0, 0)),
            pl.BlockSpec((1, 1), lambda i: (0, 0)),
        ],
        out_shape=[
            jax.ShapeDtypeStruct((1, S * BH), f32),
            jax.ShapeDtypeStruct((1, 1), f32),
        ],
        scratch_shapes=[pltpu.VMEM((1, H), f32)],
    )(acc2, deg2, W_gnn, b_gnn.reshape(1, H), ln_scale.reshape(1, H),
      ln_bias.reshape(1, H), W_pol, b_pol.reshape(1, S * BH),
      W_val, b_val.reshape(1, 1))

    logits = logits2d.reshape(1, S, BH)
    value = value2d.reshape(1)
    return (logits, value)


# speedup vs baseline: 4.4673x; 1.0008x over previous
import jax
import jax.numpy as jnp
from jax import lax
from jax.experimental import pallas as pl
from jax.experimental.pallas import tpu as pltpu
from jax.experimental.pallas import tpu_sc as plsc

N = 10000
D = 128
H = 128
S = 64
BH = 32

NC = 2
NS = 16
GB = 128
CH = 4
BLK_PER_TILE = 80
NCH = BLK_PER_TILE // CH
EDGES_PER_TILE = BLK_PER_TILE * GB
E_PAD = NC * NS * EDGES_PER_TILE
NP = 10240
ROWS_PER_TILE = NP // NS

TC_BLK = 1024
TC_STEPS = NP // TC_BLK


def _sc_edge_body(x_hbm, src_hbm, dst_hbm, zrows_hbm, zdeg_hbm,
                  out_hbm, outd_hbm,
                  src_idx, dst_idx, rows0, rows1, deg, acc, sg0, sg1, si):
    c = lax.axis_index("c")
    s = lax.axis_index("s")
    rows = (rows0, rows1)
    sg = (sg0, sg1)
    pltpu.sync_copy(zrows_hbm, acc.at[pl.ds(s * ROWS_PER_TILE, ROWS_PER_TILE)])
    pltpu.sync_copy(zdeg_hbm, deg)

    blk_base = (c * NS + s) * BLK_PER_TILE

    def _refill(buf, ch, issue):
        for hbm_ref, vref in ((src_hbm, src_idx), (dst_hbm, dst_idx)):
            cp = pltpu.make_async_copy(
                hbm_ref.at[pl.ds(blk_base + ch * CH, CH)], vref.at[buf], si)
            cp.start() if issue else cp.wait()

    ones16 = jnp.full((16,), 1.0, jnp.float32)

    pltpu.sync_copy(src_hbm.at[pl.ds(blk_base, CH)], src_idx.at[0])
    pltpu.sync_copy(dst_hbm.at[pl.ds(blk_base, CH)], dst_idx.at[0])
    _refill(1, 1, True)
    plsc.subcore_barrier()
    pltpu.async_copy(x_hbm.at[src_idx.at[0].at[0]], rows0, sg0)

    def _chunk(b, ch):
        for jj in range(CH):
            cur = jj % 2
            nxt = 1 - cur
            if jj < CH - 1:
                pltpu.async_copy(x_hbm.at[src_idx.at[b].at[jj + 1]],
                                 rows[nxt], sg[nxt])
            else:
                @pl.when(ch + 1 < NCH)
                def _pref():
                    _refill(1 - b, ch + 1, False)
                    pltpu.async_copy(x_hbm.at[src_idx.at[1 - b].at[0]],
                                     rows[nxt], sg[nxt])
            for k in range(GB // 16):
                vals = dst_idx[b, jj, pl.ds(k * 16, 16)]
                plsc.addupdate_scatter(deg, [vals], ones16)
            pltpu.make_async_copy(x_hbm.at[src_idx.at[b].at[jj]],
                                  rows[cur], sg[cur]).wait()
            pltpu.sync_copy(rows[cur], acc.at[dst_idx.at[b].at[jj]], add=True)
        @pl.when(ch + 2 < NCH)
        def _next_refill():
            _refill(b, ch + 2, True)

    @pl.loop(0, NCH, step=2)
    def _chunk_pair(ch):
        _chunk(0, ch)
        _chunk(1, ch + 1)

    plsc.subcore_barrier()
    pltpu.sync_copy(acc.at[pl.ds(s * ROWS_PER_TILE, ROWS_PER_TILE)],
                    out_hbm.at[c].at[pl.ds(s * ROWS_PER_TILE, ROWS_PER_TILE)])
    pltpu.sync_copy(deg, outd_hbm.at[c].at[s])


def _tc_tail_body(acc_ref, deg_ref, wg_ref, bg_ref, lns_ref, lnb_ref,
                  wp_ref, bp_ref, wv_ref, bv_ref,
                  logits_ref, value_ref, psum):
    i = pl.program_id(0)
    st = acc_ref[0] + acc_ref[1]
    deg = jnp.sum(deg_ref[...], axis=(0, 1)).reshape(TC_BLK, 1)
    aw = st / jnp.maximum(deg, 1.0)
    h = jnp.dot(aw, wg_ref[...], preferred_element_type=jnp.float32) + bg_ref[...]
    h = jnp.maximum(h, 0.0)
    row_id = i * TC_BLK + lax.broadcasted_iota(jnp.int32, (TC_BLK, 1), 0)
    h = jnp.where(row_id < N, h, 0.0)
    part = jnp.sum(h, axis=0, keepdims=True)

    @pl.when(i == 0)
    def _init():
        psum[...] = part

    @pl.when(i > 0)
    def _accum():
        psum[...] = psum[...] + part

    @pl.when(i == TC_STEPS - 1)
    def _finish():
        pooled = psum[...] * (1.0 / N)
        mu = jnp.mean(pooled, axis=1, keepdims=True)
        var = jnp.mean((pooled - mu) ** 2, axis=1, keepdims=True)
        nrm = (pooled - mu) / jnp.sqrt(var + 1e-5) * lns_ref[...] + lnb_ref[...]
        logits_ref[...] = (
            jnp.dot(nrm, wp_ref[...], preferred_element_type=jnp.float32)
            + bp_ref[...])
        value_ref[...] = (
            jnp.dot(nrm, wv_ref[...], preferred_element_type=jnp.float32)
            + bv_ref[...])


def kernel(x, edge_index, batch, W_gnn, b_gnn, ln_scale, ln_bias,
           W_pol, b_pol, W_val, b_val):
    f32 = jnp.float32
    n_pad = E_PAD - edge_index.shape[1]
    pad_dst = N + (jnp.arange(n_pad, dtype=jnp.int32) % 128)
    src = jnp.concatenate([edge_index[0], jnp.zeros((n_pad,), jnp.int32)])
    dst = jnp.concatenate([edge_index[1], pad_dst])
    src2d = src.reshape(NC * NS * BLK_PER_TILE, GB)
    dst2d = dst.reshape(NC * NS * BLK_PER_TILE, GB)
    zrows = jnp.zeros((ROWS_PER_TILE, H), f32)
    zdeg = jnp.zeros((NP,), f32)

    sc_edge = pl.kernel(
        _sc_edge_body,
        out_type=[
            jax.ShapeDtypeStruct((NC, NP, H), f32),
            jax.ShapeDtypeStruct((NC, NS, NP), f32),
        ],
        mesh=plsc.VectorSubcoreMesh(
            core_axis_name="c", subcore_axis_name="s",
            num_cores=NC, num_subcores=NS),
        compiler_params=pltpu.CompilerParams(needs_layout_passes=False),
        scratch_types=[
            pltpu.VMEM((2, CH, GB), jnp.int32),
            pltpu.VMEM((2, CH, GB), jnp.int32),
            pltpu.VMEM((GB, H), f32),
            pltpu.VMEM((GB, H), f32),
            pltpu.VMEM((NP,), f32),
            pltpu.VMEM_SHARED((NP, H), f32),
            pltpu.SemaphoreType.DMA,
            pltpu.SemaphoreType.DMA,
            pltpu.SemaphoreType.DMA,
        ],
    )
    acc2, deg2 = sc_edge(x, src2d, dst2d, zrows, zdeg)

    grid = (TC_STEPS,)
    logits2d, value2d = pl.pallas_call(
        _tc_tail_body,
        grid=grid,
        in_specs=[
            pl.BlockSpec((NC, TC_BLK, H), lambda i: (0, i, 0)),
            pl.BlockSpec((NC, NS, TC_BLK), lambda i: (0, 0, i)),
            pl.BlockSpec((H, H), lambda i: (0, 0)),
            pl.BlockSpec((1, H), lambda i: (0, 0)),
            pl.BlockSpec((1, H), lambda i: (0, 0)),
            pl.BlockSpec((1, H), lambda i: (0, 0)),
            pl.BlockSpec((H, S * BH), lambda i: (0, 0)),
            pl.BlockSpec((1, S * BH), lambda i: (0, 0)),
            pl.BlockSpec((H, 1), lambda i: (0, 0)),
            pl.BlockSpec((1, 1), lambda i: (0, 0)),
        ],
        out_specs=[
            pl.BlockSpec((1, S * BH), lambda i: (0, 0)),
            pl.BlockSpec((1, 1), lambda i: (0, 0)),
        ],
        out_shape=[
            jax.ShapeDtypeStruct((1, S * BH), f32),
            jax.ShapeDtypeStruct((1, 1), f32),
        ],
        scratch_shapes=[pltpu.VMEM((1, H), f32)],
    )(acc2, deg2, W_gnn, b_gnn.reshape(1, H), ln_scale.reshape(1, H),
      ln_bias.reshape(1, H), W_pol, b_pol.reshape(1, S * BH),
      W_val, b_val.reshape(1, 1))

    logits = logits2d.reshape(1, S, BH)
    value = value2d.reshape(1)
    return (logits, value)


# speedup vs baseline: 4.7974x; 1.0739x over previous
import jax
import jax.numpy as jnp
from jax import lax
from jax.experimental import pallas as pl
from jax.experimental.pallas import tpu as pltpu
from jax.experimental.pallas import tpu_sc as plsc

N = 10000
D = 128
H = 128
S = 64
BH = 32

NC = 2
NS = 16
GB = 128
CH = 4
BLK_T0 = 128
BLK_T1 = 32
NCH0 = BLK_T0 // CH
NCH1 = BLK_T1 // CH
CORE0_BLKS = NS * BLK_T0
TOT_BLKS = NS * (BLK_T0 + BLK_T1)
E_PAD = TOT_BLKS * GB
NP = 10240
ROWS_PER_TILE = NP // NS

TC_BLK = 1024
TC_STEPS = NP // TC_BLK


def _sc_edge_body(x_hbm, src_hbm, dst_hbm, zrows_hbm, zdeg_hbm,
                  out_hbm, outd_hbm,
                  src_idx, dst_idx, rows0, rows1, deg, acc, sg0, sg1, si):
    c = lax.axis_index("c")
    s = lax.axis_index("s")
    rows = (rows0, rows1)
    sg = (sg0, sg1)
    pltpu.sync_copy(zrows_hbm, acc.at[pl.ds(s * ROWS_PER_TILE, ROWS_PER_TILE)])
    pltpu.sync_copy(zdeg_hbm, deg)

    blk_base = jnp.where(c == 0, s * BLK_T0, CORE0_BLKS + s * BLK_T1)
    nch = jnp.where(c == 0, NCH0, NCH1)

    def _refill(buf, ch, issue):
        for hbm_ref, vref in ((src_hbm, src_idx), (dst_hbm, dst_idx)):
            cp = pltpu.make_async_copy(
                hbm_ref.at[pl.ds(blk_base + ch * CH, CH)], vref.at[buf], si)
            cp.start() if issue else cp.wait()

    ones16 = jnp.full((16,), 1.0, jnp.float32)

    pltpu.sync_copy(src_hbm.at[pl.ds(blk_base, CH)], src_idx.at[0])
    pltpu.sync_copy(dst_hbm.at[pl.ds(blk_base, CH)], dst_idx.at[0])
    _refill(1, 1, True)
    pltpu.async_copy(x_hbm.at[src_idx.at[0].at[0]], rows0, sg0)
    plsc.subcore_barrier()

    def _chunk(b, ch):
        for jj in range(CH):
            cur = jj % 2
            nxt = 1 - cur
            if jj < CH - 1:
                pltpu.async_copy(x_hbm.at[src_idx.at[b].at[jj + 1]],
                                 rows[nxt], sg[nxt])
            else:
                @pl.when(ch + 1 < nch)
                def _pref():
                    _refill(1 - b, ch + 1, False)
                    pltpu.async_copy(x_hbm.at[src_idx.at[1 - b].at[0]],
                                     rows[nxt], sg[nxt])
            for k in range(GB // 16):
                vals = dst_idx[b, jj, pl.ds(k * 16, 16)]
                plsc.addupdate_scatter(deg, [vals], ones16)
            pltpu.make_async_copy(x_hbm.at[src_idx.at[b].at[jj]],
                                  rows[cur], sg[cur]).wait()
            pltpu.sync_copy(rows[cur], acc.at[dst_idx.at[b].at[jj]], add=True)
        @pl.when(ch + 2 < nch)
        def _next_refill():
            _refill(b, ch + 2, True)

    @pl.loop(0, nch, step=2)
    def _chunk_pair(ch):
        _chunk(0, ch)
        _chunk(1, ch + 1)

    plsc.subcore_barrier()
    pltpu.sync_copy(acc.at[pl.ds(s * ROWS_PER_TILE, ROWS_PER_TILE)],
                    out_hbm.at[c].at[pl.ds(s * ROWS_PER_TILE, ROWS_PER_TILE)])
    pltpu.sync_copy(deg, outd_hbm.at[c].at[s])


def _tc_tail_body(acc_ref, deg_ref, wg_ref, bg_ref, lns_ref, lnb_ref,
                  wp_ref, bp_ref, wv_ref, bv_ref,
                  logits_ref, value_ref, psum):
    i = pl.program_id(0)
    st = acc_ref[0] + acc_ref[1]
    deg = jnp.sum(deg_ref[...], axis=(0, 1)).reshape(TC_BLK, 1)
    aw = st / jnp.maximum(deg, 1.0)
    h = jnp.dot(aw, wg_ref[...], preferred_element_type=jnp.float32) + bg_ref[...]
    h = jnp.maximum(h, 0.0)
    row_id = i * TC_BLK + lax.broadcasted_iota(jnp.int32, (TC_BLK, 1), 0)
    h = jnp.where(row_id < N, h, 0.0)
    part = jnp.sum(h, axis=0, keepdims=True)

    @pl.when(i == 0)
    def _init():
        psum[...] = part

    @pl.when(i > 0)
    def _accum():
        psum[...] = psum[...] + part

    @pl.when(i == TC_STEPS - 1)
    def _finish():
        pooled = psum[...] * (1.0 / N)
        mu = jnp.mean(pooled, axis=1, keepdims=True)
        var = jnp.mean((pooled - mu) ** 2, axis=1, keepdims=True)
        nrm = (pooled - mu) / jnp.sqrt(var + 1e-5) * lns_ref[...] + lnb_ref[...]
        logits_ref[...] = (
            jnp.dot(nrm, wp_ref[...], preferred_element_type=jnp.float32)
            + bp_ref[...])
        value_ref[...] = (
            jnp.dot(nrm, wv_ref[...], preferred_element_type=jnp.float32)
            + bv_ref[...])


def kernel(x, edge_index, batch, W_gnn, b_gnn, ln_scale, ln_bias,
           W_pol, b_pol, W_val, b_val):
    f32 = jnp.float32
    n_pad = E_PAD - edge_index.shape[1]
    pad_dst = N + (jnp.arange(n_pad, dtype=jnp.int32) % 128)
    src = jnp.concatenate([edge_index[0], jnp.zeros((n_pad,), jnp.int32)])
    dst = jnp.concatenate([edge_index[1], pad_dst])
    src2d = src.reshape(TOT_BLKS, GB)
    dst2d = dst.reshape(TOT_BLKS, GB)
    zrows = jnp.zeros((ROWS_PER_TILE, H), f32)
    zdeg = jnp.zeros((NP,), f32)

    sc_edge = pl.kernel(
        _sc_edge_body,
        out_type=[
            jax.ShapeDtypeStruct((NC, NP, H), f32),
            jax.ShapeDtypeStruct((NC, NS, NP), f32),
        ],
        mesh=plsc.VectorSubcoreMesh(
            core_axis_name="c", subcore_axis_name="s",
            num_cores=NC, num_subcores=NS),
        compiler_params=pltpu.CompilerParams(needs_layout_passes=False),
        scratch_types=[
            pltpu.VMEM((2, CH, GB), jnp.int32),
            pltpu.VMEM((2, CH, GB), jnp.int32),
            pltpu.VMEM((GB, H), f32),
            pltpu.VMEM((GB, H), f32),
            pltpu.VMEM((NP,), f32),
            pltpu.VMEM_SHARED((NP, H), f32),
            pltpu.SemaphoreType.DMA,
            pltpu.SemaphoreType.DMA,
            pltpu.SemaphoreType.DMA,
        ],
    )
    acc2, deg2 = sc_edge(x, src2d, dst2d, zrows, zdeg)

    grid = (TC_STEPS,)
    logits2d, value2d = pl.pallas_call(
        _tc_tail_body,
        grid=grid,
        in_specs=[
            pl.BlockSpec((NC, TC_BLK, H), lambda i: (0, i, 0)),
            pl.BlockSpec((NC, NS, TC_BLK), lambda i: (0, 0, i)),
            pl.BlockSpec((H, H), lambda i: (0, 0)),
            pl.BlockSpec((1, H), lambda i: (0, 0)),
            pl.BlockSpec((1, H), lambda i: (0, 0)),
            pl.BlockSpec((1, H), lambda i: (0, 0)),
            pl.BlockSpec((H, S * BH), lambda i: (0, 0)),
            pl.BlockSpec((1, S * BH), lambda i: (0, 0)),
            pl.BlockSpec((H, 1), lambda i: (0, 0)),
            pl.BlockSpec((1, 1), lambda i: (0, 0)),
        ],
        out_specs=[
            pl.BlockSpec((1, S * BH), lambda i: (0, 0)),
            pl.BlockSpec((1, 1), lambda i: (0, 0)),
        ],
        out_shape=[
            jax.ShapeDtypeStruct((1, S * BH), f32),
            jax.ShapeDtypeStruct((1, 1), f32),
        ],
        scratch_shapes=[pltpu.VMEM((1, H), f32)],
    )(acc2, deg2, W_gnn, b_gnn.reshape(1, H), ln_scale.reshape(1, H),
      ln_bias.reshape(1, H), W_pol, b_pol.reshape(1, S * BH),
      W_val, b_val.reshape(1, 1))

    logits = logits2d.reshape(1, S, BH)
    value = value2d.reshape(1)
    return (logits, value)


# speedup vs baseline: 4.9646x; 1.0348x over previous
import jax
import jax.numpy as jnp
from jax import lax
from jax.experimental import pallas as pl
from jax.experimental.pallas import tpu as pltpu
from jax.experimental.pallas import tpu_sc as plsc

N = 10000
D = 128
H = 128
S = 64
BH = 32

NC = 2
NS = 16
GB = 128
CH = 16
BLK_T0 = 128
BLK_T1 = 32
NCH0 = BLK_T0 // CH
NCH1 = BLK_T1 // CH
CORE0_BLKS = NS * BLK_T0
TOT_BLKS = NS * (BLK_T0 + BLK_T1)
E_PAD = TOT_BLKS * GB
NP = 10240
ROWS_PER_TILE = NP // NS

TC_BLK = 1024
TC_STEPS = NP // TC_BLK


def _sc_edge_body(x_hbm, src_hbm, dst_hbm, out_hbm, outd_hbm,
                  src_idx, dst_idx, rows0, deg, acc, sg):
    c = lax.axis_index("c")
    s = lax.axis_index("s")
    zeros16 = jnp.zeros((16,), jnp.float32)
    ones16 = jnp.full((16,), 1.0, jnp.float32)

    @pl.loop(0, GB)
    def _zrow(j):
        for k in range(H // 16):
            rows0[j, pl.ds(k * 16, 16)] = zeros16

    @pl.loop(0, NP // GB)
    def _zdeg(j):
        for k in range(GB // 16):
            deg[j, pl.ds(k * 16, 16)] = zeros16

    for r in range(ROWS_PER_TILE // GB):
        pltpu.sync_copy(rows0, acc.at[pl.ds(s * ROWS_PER_TILE + r * GB, GB)])

    blk_base = jnp.where(c == 0, s * BLK_T0, CORE0_BLKS + s * BLK_T1)
    nch = jnp.where(c == 0, NCH0, NCH1)
    plsc.subcore_barrier()

    @pl.loop(0, nch)
    def _chunk(ch):
        pltpu.sync_copy(src_hbm.at[pl.ds(blk_base + ch * CH, CH)], src_idx)
        pltpu.sync_copy(dst_hbm.at[pl.ds(blk_base + ch * CH, CH)], dst_idx)

        @pl.loop(0, CH)
        def _blk(jj):
            cp = pltpu.async_copy(x_hbm.at[src_idx.at[jj]], rows0, sg)
            for k in range(GB // 16):
                vals = dst_idx[jj, pl.ds(k * 16, 16)]
                plsc.addupdate_scatter(deg, [vals >> 7, vals & 127], ones16)
            cp.wait()
            pltpu.sync_copy(rows0, acc.at[dst_idx.at[jj]], add=True)

    plsc.subcore_barrier()
    pltpu.sync_copy(acc.at[pl.ds(s * ROWS_PER_TILE, ROWS_PER_TILE)],
                    out_hbm.at[c].at[pl.ds(s * ROWS_PER_TILE, ROWS_PER_TILE)])
    pltpu.sync_copy(deg, outd_hbm.at[c].at[s])


def _tc_tail_body(acc_ref, deg_ref, wg_ref, bg_ref, lns_ref, lnb_ref,
                  wp_ref, bp_ref, wv_ref, bv_ref,
                  logits_ref, value_ref, psum):
    i = pl.program_id(0)
    st = acc_ref[0] + acc_ref[1]
    deg = jnp.sum(deg_ref[...], axis=(0, 1)).reshape(TC_BLK, 1)
    aw = st / jnp.maximum(deg, 1.0)
    h = jnp.dot(aw, wg_ref[...], preferred_element_type=jnp.float32) + bg_ref[...]
    h = jnp.maximum(h, 0.0)
    row_id = i * TC_BLK + lax.broadcasted_iota(jnp.int32, (TC_BLK, 1), 0)
    h = jnp.where(row_id < N, h, 0.0)
    part = jnp.sum(h, axis=0, keepdims=True)

    @pl.when(i == 0)
    def _init():
        psum[...] = part

    @pl.when(i > 0)
    def _accum():
        psum[...] = psum[...] + part

    @pl.when(i == TC_STEPS - 1)
    def _finish():
        pooled = psum[...] * (1.0 / N)
        mu = jnp.mean(pooled, axis=1, keepdims=True)
        var = jnp.mean((pooled - mu) ** 2, axis=1, keepdims=True)
        nrm = (pooled - mu) / jnp.sqrt(var + 1e-5) * lns_ref[...] + lnb_ref[...]
        logits_ref[...] = (
            jnp.dot(nrm, wp_ref[...], preferred_element_type=jnp.float32)
            + bp_ref[...])
        value_ref[...] = (
            jnp.dot(nrm, wv_ref[...], preferred_element_type=jnp.float32)
            + bv_ref[...])


def kernel(x, edge_index, batch, W_gnn, b_gnn, ln_scale, ln_bias,
           W_pol, b_pol, W_val, b_val):
    f32 = jnp.float32
    n_pad = E_PAD - edge_index.shape[1]
    pad_dst = N + (jnp.arange(n_pad, dtype=jnp.int32) % 128)
    src = jnp.concatenate([edge_index[0], jnp.zeros((n_pad,), jnp.int32)])
    dst = jnp.concatenate([edge_index[1], pad_dst])
    src2d = src.reshape(TOT_BLKS, GB)
    dst2d = dst.reshape(TOT_BLKS, GB)

    sc_edge = pl.kernel(
        _sc_edge_body,
        out_type=[
            jax.ShapeDtypeStruct((NC, NP, H), f32),
            jax.ShapeDtypeStruct((NC, NS, NP // GB, GB), f32),
        ],
        mesh=plsc.VectorSubcoreMesh(
            core_axis_name="c", subcore_axis_name="s",
            num_cores=NC, num_subcores=NS),
        compiler_params=pltpu.CompilerParams(needs_layout_passes=False),
        scratch_types=[
            pltpu.VMEM((CH, GB), jnp.int32),
            pltpu.VMEM((CH, GB), jnp.int32),
            pltpu.VMEM((GB, H), f32),
            pltpu.VMEM((NP // GB, GB), f32),
            pltpu.VMEM_SHARED((NP, H), f32),
            pltpu.SemaphoreType.DMA,
        ],
    )
    acc2, deg2 = sc_edge(x, src2d, dst2d)
    deg2 = deg2.reshape(NC, NS, NP)

    grid = (TC_STEPS,)
    logits2d, value2d = pl.pallas_call(
        _tc_tail_body,
        grid=grid,
        in_specs=[
            pl.BlockSpec((NC, TC_BLK, H), lambda i: (0, i, 0)),
            pl.BlockSpec((NC, NS, TC_BLK), lambda i: (0, 0, i)),
            pl.BlockSpec((H, H), lambda i: (0, 0)),
            pl.BlockSpec((1, H), lambda i: (0, 0)),
            pl.BlockSpec((1, H), lambda i: (0, 0)),
            pl.BlockSpec((1, H), lambda i: (0, 0)),
            pl.BlockSpec((H, S * BH), lambda i: (0, 0)),
            pl.BlockSpec((1, S * BH), lambda i: (0, 0)),
            pl.BlockSpec((H, 1), lambda i: (0, 0)),
            pl.BlockSpec((1, 1), lambda i: (0, 0)),
        ],
        out_specs=[
            pl.BlockSpec((1, S * BH), lambda i: (0, 0)),
            pl.BlockSpec((1, 1), lambda i: (0, 0)),
        ],
        out_shape=[
            jax.ShapeDtypeStruct((1, S * BH), f32),
            jax.ShapeDtypeStruct((1, 1), f32),
        ],
        scratch_shapes=[pltpu.VMEM((1, H), f32)],
    )(acc2, deg2, W_gnn, b_gnn.reshape(1, H), ln_scale.reshape(1, H),
      ln_bias.reshape(1, H), W_pol, b_pol.reshape(1, S * BH),
      W_val, b_val.reshape(1, 1))

    logits = logits2d.reshape(1, S, BH)
    value = value2d.reshape(1)
    return (logits, value)
